# baseline (speedup 1.0000x reference)
import functools

import jax
import jax.numpy as jnp
from jax import lax
from jax.experimental import pallas as pl
from jax.experimental.pallas import tpu as pltpu
from jax.experimental.pallas import tpu_sc as plsc

_B = 128
_V = 100000
_VB = 128
_NFC = _V // _VB
_TAIL = _V - _NFC * _VB
_CW = _VB * _B
_NPAIR = 25


def _body(x_hbm, tgt_hbm, out_ref, tgt64_v, tflat_v, tvv_v, bufa_v, bufb_v,
          buft_v, acc_v, red_v, f1_v, f5_v, part_v, shared_i, sem0, sem1,
          gsem):
    cid = lax.axis_index("c")
    sid = lax.axis_index("s")
    rbase = cid * 64

    iota = lax.iota(jnp.int32, 16)
    zero16i = jnp.zeros((16,), jnp.int32)
    one16i = jnp.ones((16,), jnp.int32)
    zero16f = jnp.zeros((16,), jnp.float32)
    one16f = jnp.ones((16,), jnp.float32)

    pltpu.sync_copy(tgt_hbm.at[pl.ds(pl.multiple_of(rbase, 8), 64)], tgt64_v)
    for g in range(4):
        t16 = tgt64_v[pl.ds(g * 16, 16)]
        tflat_v[pl.ds(g * 16, 16)] = t16 * 128 + (rbase + g * 16) + iota
    pltpu.async_copy(x_hbm.at[tflat_v], tvv_v, gsem).wait()

    tgs = [tgt64_v[pl.ds(g * 16, 16)] for g in range(4)]
    tvs = [tvv_v[pl.ds(g * 16, 16)] for g in range(4)]

    for g in range(4):
        acc_v[pl.ds(g * 16, 16)] = zero16i

    def start(c, buf, sem):
        @pl.when(c < _NFC)
        def _():
            pltpu.async_copy(x_hbm.at[pl.ds(c * _CW, _CW)], buf, sem)

    def wait(c, buf, sem):
        @pl.when(c < _NFC)
        def _():
            pltpu.make_async_copy(x_hbm.at[pl.ds(0, _CW)], buf, sem).wait()

    def process(c, buf):
        @pl.when(c < _NFC)
        def _():
            vb0 = c * _VB

            def sub(si, accs):
                a = list(accs)
                sbase = si * 1024 + rbase
                for j in range(8):
                    jgb = jnp.broadcast_to(vb0 + si * 8 + j, (16,))
                    for g in range(4):
                        v = buf[pl.ds(sbase + (j * 128 + g * 16), 16)]
                        m = (v > tvs[g]) | ((v == tvs[g]) & (jgb < tgs[g]))
                        a[g] = a[g] + jnp.where(m, one16i, zero16i)
                return tuple(a)

            accs = lax.fori_loop(0, _VB // 8, sub,
                                 (zero16i, zero16i, zero16i, zero16i))
            for g in range(4):
                acc_v[pl.ds(g * 16, 16)] = acc_v[pl.ds(g * 16, 16)] + accs[g]

    start(sid, bufa_v, sem0)
    start(sid + 16, bufb_v, sem1)

    def pair(p, _):
        ca = sid + 32 * p
        cb = ca + 16
        wait(ca, bufa_v, sem0)
        process(ca, bufa_v)
        start(ca + 32, bufa_v, sem0)
        wait(cb, bufb_v, sem1)
        process(cb, bufb_v)
        start(cb + 32, bufb_v, sem1)
        return 0

    lax.fori_loop(0, _NPAIR, pair, 0)

    @pl.when(sid == 15)
    def _():
        tb = _NFC * _VB
        pltpu.sync_copy(x_hbm.at[pl.ds(_NFC * _CW, _TAIL * _B)], buft_v)

        def tsub(si, accs):
            a = list(accs)
            sbase = si * 1024 + rbase
            for j in range(8):
                jgb = jnp.broadcast_to(tb + si * 8 + j, (16,))
                for g in range(4):
                    v = buft_v[pl.ds(sbase + (j * 128 + g * 16), 16)]
                    m = (v > tvs[g]) | ((v == tvs[g]) & (jgb < tgs[g]))
                    a[g] = a[g] + jnp.where(m, one16i, zero16i)
            return tuple(a)

        accs = lax.fori_loop(0, _TAIL // 8, tsub,
                             (zero16i, zero16i, zero16i, zero16i))
        for g in range(4):
            acc_v[pl.ds(g * 16, 16)] = acc_v[pl.ds(g * 16, 16)] + accs[g]

    pltpu.sync_copy(acc_v, shared_i.at[sid])
    plsc.subcore_barrier()

    @pl.when(sid == 0)
    def _():
        def red(i, racc):
            pltpu.sync_copy(shared_i.at[i], red_v)
            return tuple(r + red_v[pl.ds(g * 16, 16)]
                         for g, r in enumerate(racc))

        ranks = lax.fori_loop(0, 16, red,
                              (zero16i, zero16i, zero16i, zero16i))
        f1 = zero16f
        f5 = zero16f
        for g in range(4):
            f1 = f1 + jnp.where(ranks[g] < 1, one16f, zero16f)
            f5 = f5 + jnp.where(ranks[g] < 5, one16f, zero16f)
        f1_v[...] = f1
        f5_v[...] = f5
        top1 = f1_v[pl.ds(0, 1)][0]
        top5 = f5_v[pl.ds(0, 1)][0]
        for q in range(1, 16):
            top1 = top1 + f1_v[pl.ds(q, 1)][0]
            top5 = top5 + f5_v[pl.ds(q, 1)][0]
        part_v[...] = jnp.where(iota == 0, top1,
                                jnp.where(iota == 1, top5, zero16f))
        pltpu.sync_copy(part_v, out_ref.at[cid])


@jax.jit
def _run(outputs, targets):
    x1 = outputs.T.reshape(-1)
    mesh = plsc.VectorSubcoreMesh(core_axis_name="c", subcore_axis_name="s")
    f = functools.partial(
        pl.kernel,
        mesh=mesh,
        out_type=jax.ShapeDtypeStruct((2, 16), jnp.float32),
        scratch_types=[
            pltpu.VMEM((64,), jnp.int32),
            pltpu.VMEM((64,), jnp.int32),
            pltpu.VMEM((64,), jnp.float32),
            pltpu.VMEM((_CW,), jnp.float32),
            pltpu.VMEM((_CW,), jnp.float32),
            pltpu.VMEM((_TAIL * _B,), jnp.float32),
            pltpu.VMEM((64,), jnp.int32),
            pltpu.VMEM((64,), jnp.int32),
            pltpu.VMEM((16,), jnp.float32),
            pltpu.VMEM((16,), jnp.float32),
            pltpu.VMEM((16,), jnp.float32),
            pltpu.VMEM_SHARED((16, 64), jnp.int32),
            pltpu.SemaphoreType.DMA,
            pltpu.SemaphoreType.DMA,
            pltpu.SemaphoreType.DMA,
        ],
    )(_body)
    return f(x1, targets)


def kernel(outputs, targets):
    out = _run(outputs, targets)
    s = out[0] + out[1]
    return (s[0], s[1])


# speedup vs baseline: 2.2614x; 2.2614x over previous
import functools

import jax
import jax.numpy as jnp
from jax import lax
from jax.experimental import pallas as pl
from jax.experimental.pallas import tpu as pltpu
from jax.experimental.pallas import tpu_sc as plsc

_B = 128
_V = 100000
_RB = 8
_HALF_T = 391
_HALF_C = _HALF_T * 128
_C = 2048
_NFULL = 24
_TAIL_C = _HALF_C - _NFULL * _C
_GV = 16
_NG = (_C // 16) // _GV
_NTG = (_TAIL_C // 16) // 8


def _body(x_hbm, tgt_hbm, out_ref, tgt_v, bufa_v, bufb_v, buft_v, tvblk_v,
          acc8_v, prt_v, tmp16_v, part_v, red_v, tot_v, shared_i, shared_f,
          sem0, sem1):
    cid = lax.axis_index("c")
    sid = lax.axis_index("s")
    wid = cid * 16 + sid
    rb = wid // 2
    h = wid % 2
    rbase = rb * _RB
    hs = h * _HALF_C

    pltpu.sync_copy(tgt_hbm, tgt_v)
    iota = lax.iota(jnp.int32, 16)
    zero16i = jnp.zeros((16,), jnp.int32)
    one16i = jnp.ones((16,), jnp.int32)

    ts = []
    tvs = []
    for i in range(_RB):
        ts.append(tgt_v[pl.ds(rbase + i, 1)][0])
    for i in range(_RB):
        tcol = (ts[i] // 128) * 128
        pltpu.sync_copy(
            x_hbm.at[pl.ds(pl.multiple_of(rbase, 8), _RB),
                     pl.ds(pl.multiple_of(tcol, 128), 128)],
            tvblk_v)
        tvs.append(tvblk_v[i, pl.ds(ts[i] - tcol, 1)][0])

    for i in range(_RB):
        acc8_v[pl.ds(i * 16, 16)] = zero16i

    def start(c, buf, sem):
        cs = pl.multiple_of(hs + c * _C, 128)
        pltpu.async_copy(
            x_hbm.at[pl.ds(pl.multiple_of(rbase, 8), _RB), pl.ds(cs, _C)],
            buf, sem)

    def wait(buf, sem):
        pltpu.make_async_copy(
            x_hbm.at[pl.ds(0, _RB), pl.ds(0, _C)], buf, sem).wait()

    def count4(load, pred, n):
        a = [zero16i, zero16i, zero16i, zero16i]
        for k in range(n):
            a[k % 4] = a[k % 4] + jnp.where(pred(load(k)), one16i, zero16i)
        return (a[0] + a[1]) + (a[2] + a[3])

    def process(c, buf):
        s = hs + c * _C
        e = s + _C
        for i in range(_RB):
            t_i = ts[i]
            tv_i = tvs[i]

            def grp_gt(g, acc):
                gb = g * (_GV * 16)
                return acc + count4(
                    lambda k: buf[i, pl.ds(gb + k * 16, 16)],
                    lambda v: v > tv_i, _GV)

            acc = lax.fori_loop(0, _NG, grp_gt, zero16i)
            acc8_v[pl.ds(i * 16, 16)] = acc8_v[pl.ds(i * 16, 16)] + acc

            @pl.when(e <= t_i)
            def _():
                def grp_eq(g, acc):
                    gb = g * (_GV * 16)
                    return acc + count4(
                        lambda k: buf[i, pl.ds(gb + k * 16, 16)],
                        lambda v: v == tv_i, _GV)

                acc = lax.fori_loop(0, _NG, grp_eq, zero16i)
                acc8_v[pl.ds(i * 16, 16)] = acc8_v[pl.ds(i * 16, 16)] + acc

            @pl.when((s < t_i) & (t_i < e))
            def _():
                def grp_mid(g, acc):
                    gb = g * (_GV * 16)
                    col = iota + (s + gb)
                    a = zero16i
                    for k in range(_GV):
                        v = buf[i, pl.ds(gb + k * 16, 16)]
                        m = (v == tv_i) & (col < t_i)
                        a = a + jnp.where(m, one16i, zero16i)
                        col = col + 16
                    return acc + a

                acc = lax.fori_loop(0, _NG, grp_mid, zero16i)
                acc8_v[pl.ds(i * 16, 16)] = acc8_v[pl.ds(i * 16, 16)] + acc

    start(0, bufa_v, sem0)
    start(1, bufb_v, sem1)

    def pair(p, _):
        c0 = 2 * p
        wait(bufa_v, sem0)
        process(c0, bufa_v)

        @pl.when(c0 + 2 < _NFULL)
        def _():
            start(c0 + 2, bufa_v, sem0)

        wait(bufb_v, sem1)
        process(c0 + 1, bufb_v)

        @pl.when(c0 + 3 < _NFULL)
        def _():
            start(c0 + 3, bufb_v, sem1)

        return 0

    lax.fori_loop(0, _NFULL // 2, pair, 0)

    tts = hs + _NFULL * _C
    pltpu.sync_copy(
        x_hbm.at[pl.ds(pl.multiple_of(rbase, 8), _RB),
                 pl.ds(pl.multiple_of(tts, 128), _TAIL_C)],
        buft_v)
    for i in range(_RB):
        t_i = ts[i]
        tv_i = tvs[i]

        def tgrp(g, acc):
            gb = g * 128
            col = iota + (tts + gb)
            a = zero16i
            for k in range(8):
                v = buft_v[i, pl.ds(gb + k * 16, 16)]
                m = ((v > tv_i) & (col < _V)) | ((v == tv_i) & (col < t_i))
                a = a + jnp.where(m, one16i, zero16i)
                col = col + 16
            return acc + a

        acc = lax.fori_loop(0, _NTG, tgrp, zero16i)
        acc8_v[pl.ds(i * 16, 16)] = acc8_v[pl.ds(i * 16, 16)] + acc

    pltpu.sync_copy(acc8_v, shared_i.at[sid])
    plsc.subcore_barrier()

    part_v[...] = jnp.zeros((16,), jnp.float32)

    @pl.when(sid % 2 == 0)
    def _():
        pltpu.sync_copy(shared_i.at[sid + 1], prt_v)
        top1 = jnp.float32(0.0)
        top5 = jnp.float32(0.0)
        for i in range(_RB):
            tmp16_v[...] = (acc8_v[pl.ds(i * 16, 16)]
                            + prt_v[pl.ds(i * 16, 16)])
            rank = tmp16_v[pl.ds(0, 1)][0]
            for q in range(1, 16):
                rank = rank + tmp16_v[pl.ds(q, 1)][0]
            top1 = top1 + jnp.where(rank < 1, 1.0, 0.0).astype(jnp.float32)
            top5 = top5 + jnp.where(rank < 5, 1.0, 0.0).astype(jnp.float32)
        part_v[...] = jnp.where(iota == 0, top1,
                                jnp.where(iota == 1, top5,
                                          jnp.zeros((16,), jnp.float32)))

    pltpu.sync_copy(part_v, shared_f.at[sid])
    plsc.subcore_barrier()

    @pl.when(sid == 0)
    def _():
        def red(i, a):
            pltpu.sync_copy(shared_f.at[i], red_v)
            return a + red_v[...]

        tot = lax.fori_loop(0, 16, red, jnp.zeros((16,), jnp.float32))
        tot_v[...] = tot
        pltpu.sync_copy(tot_v, out_ref.at[cid])


@jax.jit
def _run(outputs, targets):
    mesh = plsc.VectorSubcoreMesh(core_axis_name="c", subcore_axis_name="s")
    f = functools.partial(
        pl.kernel,
        mesh=mesh,
        out_type=jax.ShapeDtypeStruct((2, 16), jnp.float32),
        scratch_types=[
            pltpu.VMEM((_B,), jnp.int32),
            pltpu.VMEM((_RB, _C), jnp.float32),
            pltpu.VMEM((_RB, _C), jnp.float32),
            pltpu.VMEM((_RB, _TAIL_C), jnp.float32),
            pltpu.VMEM((_RB, 128), jnp.float32),
            pltpu.VMEM((128,), jnp.int32),
            pltpu.VMEM((128,), jnp.int32),
            pltpu.VMEM((16,), jnp.int32),
            pltpu.VMEM((16,), jnp.float32),
            pltpu.VMEM((16,), jnp.float32),
            pltpu.VMEM((16,), jnp.float32),
            pltpu.VMEM_SHARED((16, 128), jnp.int32),
            pltpu.VMEM_SHARED((16, 16), jnp.float32),
            pltpu.SemaphoreType.DMA,
            pltpu.SemaphoreType.DMA,
        ],
    )(_body)
    return f(outputs, targets)


def kernel(outputs, targets):
    out = _run(outputs, targets)
    s = out[0] + out[1]
    return (s[0], s[1])
